# Initial kernel scaffold; baseline (speedup 1.0000x reference)
import jax
import jax.numpy as jnp
from jax.experimental import pallas as pl


def kernel(nuc, elec, dist, edge_type, senders, receivers, w_same_W1, w_same_b1, w_same_W2, w_same_b2, w_anti_W1, w_anti_b1, w_anti_W2, w_anti_b2, w_n_W1, w_n_b1, w_n_W2, w_n_b2, g_same_W, g_same_b, g_anti_W, g_anti_b, g_n_W, g_n_b, h_W, h_b):
    raise NotImplementedError("write your pallas kernel here")


# speedup vs baseline: 2.1572x; 2.1572x over previous
import functools
import math

import jax
import jax.numpy as jnp
from jax import lax
from jax.experimental import pallas as pl
from jax.experimental.pallas import tpu as pltpu
from jax.experimental.pallas import tpu_sc as plsc

_NNUC = 2000
_NELEC = 48000
_NNODES = 50000
_EMB = 64
_DIN = 32
_E = 800000
_GCH = 512
_NC, _NS = 2, 16
_NW = _NC * _NS
_EPAD = 802816
_EPW = _EPAD // _NW
_EPT = _EPAD // _NS
_NPS = 24000
_NPSP = 24192
_RPT = _NPSP // _NS

_LOG_HALF = math.log(0.5)


def _ssp(x):
    return jnp.logaddexp(x, 0.0) + _LOG_HALF


def _tc_hx(elec, h_W, h_b):
    B = 480

    def body(e_ref, w_ref, b_ref, o_ref):
        o_ref[...] = (
            jnp.dot(e_ref[...], w_ref[...], preferred_element_type=jnp.float32)
            + b_ref[...]
        )

    return pl.pallas_call(
        body,
        grid=(_NELEC // B,),
        in_specs=[
            pl.BlockSpec((B, _EMB), lambda i: (i, 0)),
            pl.BlockSpec((_EMB, _EMB), lambda i: (0, 0)),
            pl.BlockSpec((1, _EMB), lambda i: (0, 0)),
        ],
        out_specs=pl.BlockSpec((B, _EMB), lambda i: (i, 0)),
        out_shape=jax.ShapeDtypeStruct((_NELEC, _EMB), jnp.float32),
    )(elec, h_W, h_b.reshape(1, _EMB))


def _tc_edge(dist_p, et2d, hs, W1c, b1c, W2v, b2n, b2s, b2a, Gv):
    B = 512

    def body(d_ref, t_ref, h_ref, w1_ref, b1_ref, w2_ref,
             bn_ref, bs_ref, ba_ref, g_ref, o_ref):
        et = t_ref[...]
        mn = (et == 1).astype(jnp.float32)
        ms = (et == 3).astype(jnp.float32)
        ma = (et == 4).astype(jnp.float32)
        h1 = _ssp(
            jnp.dot(d_ref[...], w1_ref[...], preferred_element_type=jnp.float32)
            + b1_ref[...]
        )
        h1m = jnp.concatenate(
            [h1[:, 0:32] * mn, h1[:, 32:64] * ms, h1[:, 64:96] * ma], axis=1)
        we = jnp.dot(h1m, w2_ref[...], preferred_element_type=jnp.float32)
        we = we + mn * bn_ref[...] + ms * bs_ref[...] + ma * ba_ref[...]
        weh = we * h_ref[...]
        cat = jnp.concatenate([weh * mn, weh * ms, weh * ma], axis=1)
        o_ref[...] = jnp.dot(cat, g_ref[...], preferred_element_type=jnp.float32)

    full = lambda i: (0, 0)
    return pl.pallas_call(
        body,
        grid=(_EPAD // B,),
        in_specs=[
            pl.BlockSpec((B, _DIN), lambda i: (i, 0)),
            pl.BlockSpec((B, 1), lambda i: (i, 0)),
            pl.BlockSpec((B, _EMB), lambda i: (i, 0)),
            pl.BlockSpec((_DIN, 96), full),
            pl.BlockSpec((1, 96), full),
            pl.BlockSpec((96, _EMB), full),
            pl.BlockSpec((1, _EMB), full),
            pl.BlockSpec((1, _EMB), full),
            pl.BlockSpec((1, _EMB), full),
            pl.BlockSpec((192, _EMB), full),
        ],
        out_specs=pl.BlockSpec((B, _EMB), lambda i: (i, 0)),
        out_shape=jax.ShapeDtypeStruct((_EPAD, _EMB), jnp.float32),
    )(dist_p, et2d, hs, W1c, b1c, W2v, b2n, b2s, b2a, Gv)


def _tc_out(elec, upd, bn, bs, ba):
    B = 480

    def body(e_ref, u_ref, n_ref, s_ref, a_ref, o_ref):
        o_ref[...] = e_ref[...] + u_ref[...] + n_ref[...] + s_ref[...] + a_ref[...]

    full = lambda i: (0, 0)
    return pl.pallas_call(
        body,
        grid=(_NELEC // B,),
        in_specs=[
            pl.BlockSpec((B, _EMB), lambda i: (i, 0)),
            pl.BlockSpec((B, _EMB), lambda i: (i, 0)),
            pl.BlockSpec((1, _EMB), full),
            pl.BlockSpec((1, _EMB), full),
            pl.BlockSpec((1, _EMB), full),
        ],
        out_specs=pl.BlockSpec((B, _EMB), lambda i: (i, 0)),
        out_shape=jax.ShapeDtypeStruct((_NELEC, _EMB), jnp.float32),
    )(elec, upd, bn.reshape(1, _EMB), bs.reshape(1, _EMB), ba.reshape(1, _EMB))


def _sc_gather(hx, snd2d):
    mesh = plsc.VectorSubcoreMesh(core_axis_name="c", subcore_axis_name="s")

    @functools.partial(
        pl.kernel, mesh=mesh,
        out_type=jax.ShapeDtypeStruct((_EPAD, _EMB), jnp.float32),
        compiler_params=pltpu.CompilerParams(use_tc_tiling_on_sc=False),
        scratch_types=[
            pltpu.VMEM((4, 128), jnp.int32),
            pltpu.VMEM((_GCH, _EMB), jnp.float32),
            pltpu.SemaphoreType.DMA,
        ],
    )
    def k(hx_hbm, snd_hbm, out_hbm, idx_v, rows_v, sem):
        wid = lax.axis_index("s") * _NC + lax.axis_index("c")
        rbase = wid * (_EPW // 128)
        ebase = wid * _EPW

        def body(i, carry):
            pltpu.sync_copy(snd_hbm.at[pl.ds(rbase + i * 4, 4)], idx_v)
            for j in range(4):
                pltpu.async_copy(
                    hx_hbm.at[idx_v.at[j]],
                    rows_v.at[pl.ds(j * 128, 128)], sem).wait()
            pltpu.sync_copy(rows_v, out_hbm.at[pl.ds(ebase + i * _GCH, _GCH)])
            return carry

        lax.fori_loop(0, _EPW // _GCH, body, 0)

    return k(hx, snd2d)


def _sc_scatter(veh, rcv2d, zrows):
    mesh = plsc.VectorSubcoreMesh(core_axis_name="c", subcore_axis_name="s")

    @functools.partial(
        pl.kernel, mesh=mesh,
        out_type=jax.ShapeDtypeStruct((2 * _NPSP, _EMB), jnp.float32),
        compiler_params=pltpu.CompilerParams(use_tc_tiling_on_sc=False),
        scratch_types=[
            pltpu.VMEM((_GCH, _EMB), jnp.float32),
            pltpu.VMEM((4, 128), jnp.int32),
            pltpu.VMEM((4, 128), jnp.int32),
            pltpu.VMEM_SHARED((_NPSP, _EMB), jnp.float32),
            pltpu.SemaphoreType.DMA,
        ],
    )
    def k(veh_hbm, rcv_hbm, z_hbm, out_hbm, vbuf, rbuf, ibuf, acc, sem):
        c = lax.axis_index("c")
        s = lax.axis_index("s")
        nbase = _NNUC + c * _NPS
        pltpu.sync_copy(z_hbm, acc.at[pl.ds(s * _RPT, _RPT)])
        plsc.subcore_barrier()

        rtile = s * (_EPT // 128)
        etile = s * _EPT

        def body(i, carry):
            pltpu.sync_copy(rcv_hbm.at[pl.ds(rtile + i * 4, 4)], rbuf)
            pltpu.sync_copy(veh_hbm.at[pl.ds(etile + i * _GCH, _GCH)], vbuf)
            for j in range(4):
                for t in range(8):
                    r = rbuf[j, pl.ds(t * 16, 16)]
                    rr = r - nbase
                    m = (rr >= 0) & (rr < _NPS)
                    ibuf[j, pl.ds(t * 16, 16)] = jnp.where(m, rr, _NPS)
            for j in range(4):
                pltpu.sync_copy(
                    vbuf.at[pl.ds(j * 128, 128)],
                    acc.at[ibuf.at[j]], add=True)
            return carry

        lax.fori_loop(0, _EPT // _GCH, body, 0)
        plsc.subcore_barrier()
        pltpu.sync_copy(
            acc.at[pl.ds(s * _RPT, _RPT)],
            out_hbm.at[pl.ds(c * _NPSP + s * _RPT, _RPT)])

    return k(veh, rcv2d, zrows)


def kernel(nuc, elec, dist, edge_type, senders, receivers,
           w_same_W1, w_same_b1, w_same_W2, w_same_b2,
           w_anti_W1, w_anti_b1, w_anti_W2, w_anti_b2,
           w_n_W1, w_n_b1, w_n_W2, w_n_b2,
           g_same_W, g_same_b, g_anti_W, g_anti_b, g_n_W, g_n_b,
           h_W, h_b):
    pad = _EPAD - _E

    hx = jnp.concatenate([nuc, _tc_hx(elec, h_W, h_b)], axis=0)

    snd = jnp.pad(senders.astype(jnp.int32), (0, pad)).reshape(_EPAD // 128, 128)
    hs = _sc_gather(hx, snd)

    dist_p = jnp.pad(dist, ((0, pad), (0, 0)))
    et2d = jnp.pad(edge_type.astype(jnp.int32), (0, pad)).reshape(_EPAD, 1)
    W1c = jnp.concatenate([w_n_W1, w_same_W1, w_anti_W1], axis=1)
    b1c = jnp.concatenate([w_n_b1, w_same_b1, w_anti_b1]).reshape(1, 96)
    W2v = jnp.concatenate([w_n_W2, w_same_W2, w_anti_W2], axis=0)
    Gv = jnp.concatenate([g_n_W, g_same_W, g_anti_W], axis=0)
    veh = _tc_edge(dist_p, et2d, hs, W1c, b1c, W2v,
                   w_n_b2.reshape(1, _EMB), w_same_b2.reshape(1, _EMB),
                   w_anti_b2.reshape(1, _EMB), Gv)

    rcv = jnp.pad(receivers.astype(jnp.int32), (0, pad)).reshape(_EPAD // 128, 128)
    zrows = jnp.zeros((_RPT, _EMB), jnp.float32)
    accf = _sc_scatter(veh, rcv, zrows)

    upd = jnp.concatenate(
        [accf[0:_NPS], accf[_NPSP:_NPSP + _NPS]], axis=0)
    return _tc_out(elec, upd, g_n_b, g_same_b, g_anti_b)
